# baseline (speedup 1.0000x reference)
import functools

import jax
import jax.numpy as jnp
from jax import lax
from jax.experimental import pallas as pl
from jax.experimental.pallas import tpu as pltpu
from jax.experimental.pallas import tpu_sc as plsc

_NUM_EMB = 1000000
_SPLIT = 499968
_D = 64
_BATCH = 16384
_FIELDS = 26
_B = _BATCH * _FIELDS
_NC = 2
_NS = 16
_NW = _NC * _NS
_BPW = _B // _NW
_CH = 512
_NCH = _BPW // _CH
_NBUF = 2
_NG = _NCH // _NBUF


def _emb_body(idxa_hbm, idxb_hbm, taba_hbm, tabb_hbm, out_hbm,
              idxa_v, idxb_v, *rest):
    bufs = rest[:_NBUF]
    gsems_a = rest[_NBUF:2 * _NBUF]
    gsems_b = rest[2 * _NBUF:3 * _NBUF]
    psems = rest[3 * _NBUF:4 * _NBUF]

    wid = lax.axis_index("s") * _NC + lax.axis_index("c")
    base = wid * _BPW

    pltpu.sync_copy(idxa_hbm.at[wid], idxa_v)
    pltpu.sync_copy(idxb_hbm.at[wid], idxb_v)

    def gather_a(j, b):
        return pltpu.make_async_copy(
            taba_hbm.at[plsc.Indices(idxa_v.at[j], ignored_value=-1)],
            bufs[b], gsems_a[b])

    def gather_b(j, b):
        return pltpu.make_async_copy(
            tabb_hbm.at[plsc.Indices(idxb_v.at[j], ignored_value=-1)],
            bufs[b], gsems_b[b])

    def put(j, b):
        return pltpu.make_async_copy(
            bufs[b], out_hbm.at[pl.ds(base + j * _CH, _CH)], psems[b])

    def start_gathers(j, b):
        gather_a(j, b).start()
        gather_b(j, b).start()

    def wait_gathers(j, b):
        gather_a(j, b).wait()
        gather_b(j, b).wait()

    for b in range(_NBUF):
        start_gathers(b, b)

    def group(g, carry):
        for b in range(_NBUF):
            j = g * _NBUF + b
            wait_gathers(j, b)
            put(j, b).start()
            put(j, b).wait()
            start_gathers(j + _NBUF, b)
        return carry

    lax.fori_loop(0, _NG - 1, group, 0)

    for b in range(_NBUF):
        j = (_NG - 1) * _NBUF + b
        wait_gathers(j, b)
        put(j, b).start()
    for b in range(_NBUF):
        j = (_NG - 1) * _NBUF + b
        put(j, b).wait()


_emb = functools.partial(
    pl.kernel,
    out_type=jax.ShapeDtypeStruct((_B, _D), jnp.float32),
    mesh=plsc.VectorSubcoreMesh(core_axis_name="c", subcore_axis_name="s"),
    scratch_types=[
        pltpu.VMEM((_NCH, _CH), jnp.int32),
        pltpu.VMEM((_NCH, _CH), jnp.int32),
        *[pltpu.VMEM((_CH, _D), jnp.float32) for _ in range(_NBUF)],
        *[pltpu.SemaphoreType.DMA for _ in range(3 * _NBUF)],
    ],
    compiler_params=pltpu.CompilerParams(use_tc_tiling_on_sc=False),
)(_emb_body)


@jax.jit
def kernel(idx, weight):
    idx32 = idx.astype(jnp.int32)
    in_a = idx32 < _SPLIT
    idx_a = jnp.where(in_a, idx32, -1).reshape(_NW, _NCH, _CH)
    idx_b = jnp.where(in_a, -1, idx32 - _SPLIT).reshape(_NW, _NCH, _CH)
    w_a = weight[:_SPLIT]
    w_b = weight[_SPLIT:]
    out = _emb(idx_a, idx_b, w_a, w_b)
    return out.reshape(_BATCH, _FIELDS, _D)


# speedup vs baseline: 1.1160x; 1.1160x over previous
import functools

import jax
import jax.numpy as jnp
from jax import lax
from jax.experimental import pallas as pl
from jax.experimental.pallas import tpu as pltpu
from jax.experimental.pallas import tpu_sc as plsc

_NUM_EMB = 1000000
_D = 64
_BATCH = 16384
_FIELDS = 26
_B = _BATCH * _FIELDS
_NC = 2
_NS = 16
_NW = _NC * _NS
_BPW = _B // _NW
_CH = 512
_NCH = _BPW // _CH
_NBUF = 2
_NG = _NCH // _NBUF


def _emb_body(idx_hbm, table_hbm, out_hbm, idx_v, *rest):
    bufs = rest[:_NBUF]
    gsems = rest[_NBUF:2 * _NBUF]
    psems = rest[2 * _NBUF:3 * _NBUF]

    wid = lax.axis_index("s") * _NC + lax.axis_index("c")
    base = wid * _BPW

    pltpu.sync_copy(idx_hbm.at[wid], idx_v)

    def start_gather(j, b):
        pltpu.async_copy(table_hbm.at[idx_v.at[j]], bufs[b], gsems[b])

    def wait_gather(j, b):
        pltpu.make_async_copy(table_hbm.at[idx_v.at[j]], bufs[b],
                              gsems[b]).wait()

    def start_put(j, b):
        pltpu.async_copy(bufs[b], out_hbm.at[pl.ds(base + j * _CH, _CH)],
                         psems[b])

    def wait_put(j, b):
        pltpu.make_async_copy(bufs[b], out_hbm.at[pl.ds(base + j * _CH, _CH)],
                              psems[b]).wait()

    for b in range(_NBUF):
        start_gather(b, b)

    def group(g, carry):
        for b in range(_NBUF):
            j = g * _NBUF + b
            wait_gather(j, b)
            start_put(j, b)
            wait_put(j, b)
            start_gather(j + _NBUF, b)
        return carry

    lax.fori_loop(0, _NG - 1, group, 0)

    for b in range(_NBUF):
        j = (_NG - 1) * _NBUF + b
        wait_gather(j, b)
        start_put(j, b)
    for b in range(_NBUF):
        j = (_NG - 1) * _NBUF + b
        wait_put(j, b)


_emb = functools.partial(
    pl.kernel,
    out_type=jax.ShapeDtypeStruct((_B, _D), jnp.float32),
    mesh=plsc.VectorSubcoreMesh(core_axis_name="c", subcore_axis_name="s"),
    scratch_types=[
        pltpu.VMEM((_NCH, _CH), jnp.int32),
        *[pltpu.VMEM((_CH, _D), jnp.float32) for _ in range(_NBUF)],
        *[pltpu.SemaphoreType.DMA for _ in range(2 * _NBUF)],
    ],
    compiler_params=pltpu.CompilerParams(use_tc_tiling_on_sc=False),
)(_emb_body)


@jax.jit
def kernel(idx, weight):
    idx_r = idx.astype(jnp.int32).reshape(_NW, _NCH, _CH)
    out = _emb(idx_r, weight)
    return out.reshape(_BATCH, _FIELDS, _D)
